# baseline (speedup 1.0000x reference)
import functools

import jax
import jax.numpy as jnp
from jax import lax
from jax.experimental import pallas as pl
from jax.experimental.pallas import tpu as pltpu

_N_GRAPHS = 128


def _ceil_to(a, b):
    return (a + b - 1) // b * b


def _relu(v):
    return jnp.maximum(v, 0.0)


def _leaky(v):
    return jnp.where(v > 0, v, 0.01 * v)


def _dense_mm(pairs, bias=None, add=None, mul=None, act=None, bm=256, bn=256,
              bk=512):
    M, K = pairs[0][0].shape
    N = pairs[0][1].shape[1]
    bm = min(bm, M)
    bk = min(bk, K)
    nk = K // bk
    n_pairs = len(pairs)
    has_bias = bias is not None
    has_add = add is not None
    has_mul = mul is not None

    def body(*refs):
        o_ref = refs[-1]
        k = pl.program_id(2)
        acc = jnp.zeros((bm, bn), jnp.float32)
        for p in range(n_pairs):
            a_ref = refs[2 * p]
            b_ref = refs[2 * p + 1]
            acc = acc + jnp.dot(a_ref[...], b_ref[...],
                                preferred_element_type=jnp.float32)

        @pl.when(k == 0)
        def _():
            o_ref[...] = jnp.zeros_like(o_ref)

        o_ref[...] += acc

        @pl.when(k == nk - 1)
        def _():
            v = o_ref[...]
            i_extra = 2 * n_pairs
            if has_add:
                v = v + refs[i_extra][...]
                i_extra += 1
            if has_bias:
                v = v + refs[i_extra][0:1, :]
                i_extra += 1
            if act is not None:
                v = act(v)
            if has_mul:
                v = v * refs[i_extra][...]
            o_ref[...] = v

    in_specs = []
    operands = []
    for (a, b) in pairs:
        in_specs.append(pl.BlockSpec((bm, bk), lambda i, j, k: (i, k)))
        in_specs.append(pl.BlockSpec((bk, bn), lambda i, j, k: (k, j)))
        operands += [a, b]
    if has_add:
        in_specs.append(pl.BlockSpec((bm, bn), lambda i, j, k: (i, j)))
        operands.append(add)
    if has_bias:
        in_specs.append(pl.BlockSpec((8, bn), lambda i, j, k: (0, j)))
        operands.append(jnp.broadcast_to(bias.reshape(1, -1), (8, N)))
    if has_mul:
        in_specs.append(pl.BlockSpec((bm, bn), lambda i, j, k: (i, j)))
        operands.append(mul)

    return pl.pallas_call(
        body,
        grid=(M // bm, N // bn, nk),
        in_specs=in_specs,
        out_specs=pl.BlockSpec((bm, bn), lambda i, j, k: (i, j)),
        out_shape=jax.ShapeDtypeStruct((M, N), jnp.float32),
    )(*operands)


def _onehot_mm(idx_b, x, mode, m_out, add=None, mul=None, act=None,
               bm=256, bn=256, bk=256):
    KROWS, N = x.shape
    bm = min(bm, m_out)
    bk = min(bk, KROWS)
    nk = KROWS // bk
    has_add = add is not None
    has_mul = mul is not None

    def body(*refs):
        idx_ref = refs[0]
        x_ref = refs[1]
        o_ref = refs[-1]
        i = pl.program_id(0)
        k = pl.program_id(2)
        idx = idx_ref[0, 0, :]
        if mode == 'gather':
            oh = (idx[:, None] ==
                  (lax.broadcasted_iota(jnp.int32, (bm, bk), 1) + k * bk)
                  ).astype(jnp.float32)
        else:
            oh = ((lax.broadcasted_iota(jnp.int32, (bm, bk), 0) + i * bm) ==
                  idx[None, :]).astype(jnp.float32)
        acc = jnp.dot(oh, x_ref[...], preferred_element_type=jnp.float32)

        @pl.when(k == 0)
        def _():
            o_ref[...] = jnp.zeros_like(o_ref)

        o_ref[...] += acc

        @pl.when(k == nk - 1)
        def _():
            v = o_ref[...]
            i_extra = 2
            if has_add:
                v = v + refs[i_extra][...]
                i_extra += 1
            if act is not None:
                v = act(v)
            if has_mul:
                v = v * refs[i_extra][...]
            o_ref[...] = v

    if mode == 'gather':
        idx_spec = pl.BlockSpec((1, 1, bm), lambda i, j, k: (i, 0, 0))
    else:
        idx_spec = pl.BlockSpec((1, 1, bk), lambda i, j, k: (k, 0, 0))
    in_specs = [idx_spec,
                pl.BlockSpec((bk, bn), lambda i, j, k: (k, j))]
    operands = [idx_b, x]
    if has_add:
        in_specs.append(pl.BlockSpec((bm, bn), lambda i, j, k: (i, j)))
        operands.append(add)
    if has_mul:
        in_specs.append(pl.BlockSpec((bm, bn), lambda i, j, k: (i, j)))
        operands.append(mul)

    return pl.pallas_call(
        body,
        grid=(m_out // bm, N // bn, nk),
        in_specs=in_specs,
        out_specs=pl.BlockSpec((bm, bn), lambda i, j, k: (i, j)),
        out_shape=jax.ShapeDtypeStruct((m_out, N), jnp.float32),
    )(*operands)


def _pool_mean(r, gid_b, n_graphs, bn=256, bk=256):
    NP, N = r.shape
    bk = min(bk, NP)
    nk = NP // bk
    G = n_graphs

    def body(gid_ref, r_ref, o_ref, cnt_ref):
        k = pl.program_id(1)
        gid = gid_ref[0, 0, :]
        oh = (lax.broadcasted_iota(jnp.int32, (G, bk), 0) ==
              gid[None, :]).astype(jnp.float32)
        acc = jnp.dot(oh, r_ref[...], preferred_element_type=jnp.float32)
        cnt = jnp.sum(oh, axis=1, keepdims=True)

        @pl.when(k == 0)
        def _():
            o_ref[...] = jnp.zeros_like(o_ref)
            cnt_ref[...] = jnp.zeros_like(cnt_ref)

        o_ref[...] += acc
        cnt_ref[...] += jnp.broadcast_to(cnt, cnt_ref.shape)

        @pl.when(k == nk - 1)
        def _():
            c = cnt_ref[...][:, 0:1]
            o_ref[...] = o_ref[...] / jnp.maximum(c, 1.0)

    return pl.pallas_call(
        body,
        grid=(N // bn, nk),
        in_specs=[pl.BlockSpec((1, 1, bk), lambda j, k: (k, 0, 0)),
                  pl.BlockSpec((bk, bn), lambda j, k: (k, j))],
        out_specs=pl.BlockSpec((G, bn), lambda j, k: (0, j)),
        out_shape=jax.ShapeDtypeStruct((G, N), jnp.float32),
        scratch_shapes=[pltpu.VMEM((G, 128), jnp.float32)],
    )(gid_b, r)


def kernel(x, edge_index, edge_attr, graph_ids, W_in, b_in, W_msg, b_msg,
           W_node, b_node, W_pn, W_pe, W_ps, W_r1, b_r1, W_r2, b_r2):
    N, DA = x.shape
    E, DB = edge_attr.shape
    D = W_in.shape[1]
    G = _N_GRAPHS
    n_layers = 4

    NP = _ceil_to(N, 256)
    EP = _ceil_to(E, 256)
    KA = _ceil_to(DA, 128)
    KB = _ceil_to(DB, 128)

    x_p = jnp.pad(x, ((0, NP - N), (0, KA - DA)))
    W_in_p = jnp.pad(W_in, ((0, KA - DA), (0, 0)))
    ea_p = jnp.pad(edge_attr, ((0, EP - E), (0, KB - DB)))
    Wm1 = W_msg[:D]
    Wm2_p = jnp.pad(W_msg[D:], ((0, KB - DB), (0, 0)))
    W_pe_p = jnp.pad(W_pe, ((0, KB - DB), (0, 0)))
    Wn1 = W_node[:D]
    Wn2 = W_node[D:]

    src = jnp.pad(edge_index[0], (0, EP - E), constant_values=0)
    dst = jnp.pad(edge_index[1], (0, EP - E), constant_values=-1)
    src_b = src.reshape(EP // 256, 1, 256)
    dst_b = dst.reshape(EP // 256, 1, 256)
    gid_b = jnp.pad(graph_ids, (0, NP - N),
                    constant_values=G).reshape(NP // 256, 1, 256)

    h = _dense_mm([(x_p, W_in_p)], bias=b_in, act=_relu, bk=KA)
    e_msg = _dense_mm([(ea_p, Wm2_p)], bias=b_msg, bk=KB)
    he = _dense_mm([(ea_p, W_pe_p)], bk=KB)

    for _ in range(n_layers):
        hm = _dense_mm([(h, Wm1)])
        m = _onehot_mm(src_b, hm, 'gather', EP, add=e_msg, act=_relu)
        agg = _onehot_mm(dst_b, m, 'scatter', NP)
        h = _dense_mm([(h, Wn1), (agg, Wn2)], bias=b_node, act=_relu)

    hv = _dense_mm([(h, W_pn)])
    hs = _dense_mm([(h, W_ps)])
    mp = _onehot_mm(src_b, hv, 'gather', EP, mul=he)
    node_out = _onehot_mm(dst_b, mp, 'scatter', NP, mul=hs)
    r = _dense_mm([(node_out, W_r1)], bias=b_r1, act=_leaky)
    pooled = _pool_mean(r, gid_b, G)
    out = _dense_mm([(pooled, W_r2)], bias=b_r2, bm=128)
    return out


# speedup vs baseline: 1.1235x; 1.1235x over previous
import functools

import jax
import jax.numpy as jnp
from jax import lax
from jax.experimental import pallas as pl
from jax.experimental.pallas import tpu as pltpu

_N_GRAPHS = 128


def _ceil_to(a, b):
    return (a + b - 1) // b * b


def _relu(v):
    return jnp.maximum(v, 0.0)


def _leaky(v):
    return jnp.where(v > 0, v, 0.01 * v)


def _dense_mm(pairs, bias=None, add=None, mul=None, act=None, bm=512, bn=512,
              bk=512):
    M, K = pairs[0][0].shape
    N = pairs[0][1].shape[1]
    bm = min(bm, M)
    bk = min(bk, K)
    nk = K // bk
    n_pairs = len(pairs)
    has_bias = bias is not None
    has_add = add is not None
    has_mul = mul is not None

    def body(*refs):
        o_ref = refs[-1]
        k = pl.program_id(2)
        acc = jnp.zeros((bm, bn), jnp.float32)
        for p in range(n_pairs):
            a_ref = refs[2 * p]
            b_ref = refs[2 * p + 1]
            acc = acc + jnp.dot(a_ref[...].astype(jnp.bfloat16),
                                b_ref[...].astype(jnp.bfloat16),
                                preferred_element_type=jnp.float32)

        @pl.when(k == 0)
        def _():
            o_ref[...] = jnp.zeros_like(o_ref)

        o_ref[...] += acc

        @pl.when(k == nk - 1)
        def _():
            v = o_ref[...]
            i_extra = 2 * n_pairs
            if has_add:
                v = v + refs[i_extra][...]
                i_extra += 1
            if has_bias:
                v = v + refs[i_extra][0:1, :]
                i_extra += 1
            if act is not None:
                v = act(v)
            if has_mul:
                v = v * refs[i_extra][...]
            o_ref[...] = v

    in_specs = []
    operands = []
    for (a, b) in pairs:
        in_specs.append(pl.BlockSpec((bm, bk), lambda i, j, k: (i, k)))
        in_specs.append(pl.BlockSpec((bk, bn), lambda i, j, k: (k, j)))
        operands += [a, b]
    if has_add:
        in_specs.append(pl.BlockSpec((bm, bn), lambda i, j, k: (i, j)))
        operands.append(add)
    if has_bias:
        in_specs.append(pl.BlockSpec((8, bn), lambda i, j, k: (0, j)))
        operands.append(jnp.broadcast_to(bias.reshape(1, -1), (8, N)))
    if has_mul:
        in_specs.append(pl.BlockSpec((bm, bn), lambda i, j, k: (i, j)))
        operands.append(mul)

    return pl.pallas_call(
        body,
        grid=(M // bm, N // bn, nk),
        in_specs=in_specs,
        out_specs=pl.BlockSpec((bm, bn), lambda i, j, k: (i, j)),
        out_shape=jax.ShapeDtypeStruct((M, N), jnp.float32),
    )(*operands)


def _onehot_mm(idx_b, x, mode, m_out, add=None, mul=None, act=None,
               bm=256, bn=256, bk=256):
    KROWS, N = x.shape
    bm = min(bm, m_out)
    bk = min(bk, KROWS)
    nk = KROWS // bk
    has_add = add is not None
    has_mul = mul is not None

    def body(*refs):
        idx_ref = refs[0]
        x_ref = refs[1]
        o_ref = refs[-1]
        i = pl.program_id(0)
        k = pl.program_id(2)
        idx = idx_ref[0, 0, :]
        if mode == 'gather':
            oh = (idx[:, None] ==
                  (lax.broadcasted_iota(jnp.int32, (bm, bk), 1) + k * bk)
                  ).astype(jnp.bfloat16)
        else:
            oh = ((lax.broadcasted_iota(jnp.int32, (bm, bk), 0) + i * bm) ==
                  idx[None, :]).astype(jnp.bfloat16)
        acc = jnp.dot(oh, x_ref[...].astype(jnp.bfloat16),
                      preferred_element_type=jnp.float32)

        @pl.when(k == 0)
        def _():
            o_ref[...] = jnp.zeros_like(o_ref)

        o_ref[...] += acc

        @pl.when(k == nk - 1)
        def _():
            v = o_ref[...]
            i_extra = 2
            if has_add:
                v = v + refs[i_extra][...]
                i_extra += 1
            if act is not None:
                v = act(v)
            if has_mul:
                v = v * refs[i_extra][...]
            o_ref[...] = v

    if mode == 'gather':
        idx_spec = pl.BlockSpec((1, 1, bm), lambda i, j, k: (i, 0, 0))
    else:
        idx_spec = pl.BlockSpec((1, 1, bk), lambda i, j, k: (k, 0, 0))
    in_specs = [idx_spec,
                pl.BlockSpec((bk, bn), lambda i, j, k: (k, j))]
    operands = [idx_b, x]
    if has_add:
        in_specs.append(pl.BlockSpec((bm, bn), lambda i, j, k: (i, j)))
        operands.append(add)
    if has_mul:
        in_specs.append(pl.BlockSpec((bm, bn), lambda i, j, k: (i, j)))
        operands.append(mul)

    return pl.pallas_call(
        body,
        grid=(m_out // bm, N // bn, nk),
        in_specs=in_specs,
        out_specs=pl.BlockSpec((bm, bn), lambda i, j, k: (i, j)),
        out_shape=jax.ShapeDtypeStruct((m_out, N), jnp.float32),
    )(*operands)


def _pool_mean(r, gid_b, n_graphs, bn=256, bk=256):
    NP, N = r.shape
    bk = min(bk, NP)
    nk = NP // bk
    G = n_graphs

    def body(gid_ref, r_ref, o_ref, cnt_ref):
        k = pl.program_id(1)
        gid = gid_ref[0, 0, :]
        oh = (lax.broadcasted_iota(jnp.int32, (G, bk), 0) ==
              gid[None, :]).astype(jnp.float32)
        acc = jnp.dot(oh.astype(jnp.bfloat16), r_ref[...].astype(jnp.bfloat16),
                      preferred_element_type=jnp.float32)
        cnt = jnp.sum(oh, axis=1, keepdims=True)

        @pl.when(k == 0)
        def _():
            o_ref[...] = jnp.zeros_like(o_ref)
            cnt_ref[...] = jnp.zeros_like(cnt_ref)

        o_ref[...] += acc
        cnt_ref[...] += jnp.broadcast_to(cnt, cnt_ref.shape)

        @pl.when(k == nk - 1)
        def _():
            c = cnt_ref[...][:, 0:1]
            o_ref[...] = o_ref[...] / jnp.maximum(c, 1.0)

    return pl.pallas_call(
        body,
        grid=(N // bn, nk),
        in_specs=[pl.BlockSpec((1, 1, bk), lambda j, k: (k, 0, 0)),
                  pl.BlockSpec((bk, bn), lambda j, k: (k, j))],
        out_specs=pl.BlockSpec((G, bn), lambda j, k: (0, j)),
        out_shape=jax.ShapeDtypeStruct((G, N), jnp.float32),
        scratch_shapes=[pltpu.VMEM((G, 128), jnp.float32)],
    )(gid_b, r)


def kernel(x, edge_index, edge_attr, graph_ids, W_in, b_in, W_msg, b_msg,
           W_node, b_node, W_pn, W_pe, W_ps, W_r1, b_r1, W_r2, b_r2):
    N, DA = x.shape
    E, DB = edge_attr.shape
    D = W_in.shape[1]
    G = _N_GRAPHS
    n_layers = 4

    NP = _ceil_to(N, 256)
    EP = _ceil_to(E, 256)
    KA = _ceil_to(DA, 128)
    KB = _ceil_to(DB, 128)

    x_p = jnp.pad(x, ((0, NP - N), (0, KA - DA)))
    W_in_p = jnp.pad(W_in, ((0, KA - DA), (0, 0)))
    ea_p = jnp.pad(edge_attr, ((0, EP - E), (0, KB - DB)))
    Wm1 = W_msg[:D]
    Wm2_p = jnp.pad(W_msg[D:], ((0, KB - DB), (0, 0)))
    W_pe_p = jnp.pad(W_pe, ((0, KB - DB), (0, 0)))
    Wn1 = W_node[:D]
    Wn2 = W_node[D:]

    src = jnp.pad(edge_index[0], (0, EP - E), constant_values=0)
    dst = jnp.pad(edge_index[1], (0, EP - E), constant_values=-1)
    src_b = src.reshape(EP // 256, 1, 256)
    dst_b = dst.reshape(EP // 256, 1, 256)
    gid_b = jnp.pad(graph_ids, (0, NP - N),
                    constant_values=G).reshape(NP // 256, 1, 256)

    h = _dense_mm([(x_p, W_in_p)], bias=b_in, act=_relu, bk=KA)
    e_msg = _dense_mm([(ea_p, Wm2_p)], bias=b_msg, bk=KB)
    he = _dense_mm([(ea_p, W_pe_p)], bk=KB)

    for _ in range(n_layers):
        hm = _dense_mm([(h, Wm1)])
        m = _onehot_mm(src_b, hm, 'gather', EP, add=e_msg, act=_relu)
        agg = _onehot_mm(dst_b, m, 'scatter', NP)
        h = _dense_mm([(h, Wn1), (agg, Wn2)], bias=b_node, act=_relu)

    hv = _dense_mm([(h, W_pn)])
    hs = _dense_mm([(h, W_ps)])
    mp = _onehot_mm(src_b, hv, 'gather', EP, mul=he)
    node_out = _onehot_mm(dst_b, mp, 'scatter', NP, mul=hs)
    r = _dense_mm([(node_out, W_r1)], bias=b_r1, act=_leaky)
    pooled = _pool_mean(r, gid_b, G)
    out = _dense_mm([(pooled, W_r2)], bias=b_r2, bm=128)
    return out


# speedup vs baseline: 5.9343x; 5.2821x over previous
import jax
import jax.numpy as jnp
from jax import lax
from jax.experimental import pallas as pl
from jax.experimental.pallas import tpu as pltpu
from jax.experimental.pallas import tpu_sc as plsc

_N_GRAPHS = 128
_NC = 2
_NS = 16
_W = 256


def _ceil_to(a, b):
    return (a + b - 1) // b * b


def _relu(v):
    return jnp.maximum(v, 0.0)


def _leaky(v):
    return jnp.where(v > 0, v, 0.01 * v)


def _dense_mm(pairs, bias=None, add=None, mul=None, act=None, bm=512, bn=512,
              bk=512, out_blocked=False):
    if any(p[2] for p in pairs):
        bk = 256
    a0 = pairs[0][0]
    M = a0.shape[1] if pairs[0][2] else a0.shape[0]
    K = a0.shape[0] * 256 if pairs[0][2] else a0.shape[1]
    N = pairs[0][1].shape[1]
    bm = min(bm, M)
    bk = min(bk, K)
    if out_blocked:
        bn = 256
    bn = min(bn, N)
    nk = K // bk
    has_bias = bias is not None
    has_add = add is not None
    has_mul = mul is not None

    def body(*refs):
        o_ref = refs[-1]
        k = pl.program_id(2)
        acc = jnp.zeros((bm, bn), jnp.float32)
        idx = 0
        for (_, _, blocked, amul) in pairs:
            a_ref = refs[idx]
            idx += 1
            b_ref = refs[idx]
            idx += 1
            a = a_ref[0] if blocked else a_ref[...]
            if amul is not None:
                a = a * refs[idx][...]
                idx += 1
            acc = acc + jnp.dot(a.astype(jnp.bfloat16),
                                b_ref[...].astype(jnp.bfloat16),
                                preferred_element_type=jnp.float32)
        n_in = idx

        def store(v):
            if out_blocked:
                o_ref[0] = v
            else:
                o_ref[...] = v

        def load():
            return o_ref[0] if out_blocked else o_ref[...]

        @pl.when(k == 0)
        def _():
            store(jnp.zeros((bm, bn), jnp.float32))

        store(load() + acc)

        @pl.when(k == nk - 1)
        def _():
            v = load()
            i_extra = n_in
            if has_add:
                v = v + refs[i_extra][...]
                i_extra += 1
            if has_bias:
                v = v + refs[i_extra][0:1, :]
                i_extra += 1
            if act is not None:
                v = act(v)
            if has_mul:
                v = v * refs[i_extra][...]
            store(v)

    in_specs = []
    operands = []
    for (a, b, blocked, amul) in pairs:
        if blocked:
            in_specs.append(
                pl.BlockSpec((1, bm, 256), lambda i, j, k: (k, i, 0)))
        else:
            in_specs.append(pl.BlockSpec((bm, bk), lambda i, j, k: (i, k)))
        in_specs.append(pl.BlockSpec((bk, bn), lambda i, j, k: (k, j)))
        operands += [a, b]
        if amul is not None:
            in_specs.append(pl.BlockSpec((bm, bk), lambda i, j, k: (i, k)))
            operands.append(amul)
    if has_add:
        in_specs.append(pl.BlockSpec((bm, bn), lambda i, j, k: (i, j)))
        operands.append(add)
    if has_bias:
        in_specs.append(pl.BlockSpec((8, bn), lambda i, j, k: (0, j)))
        operands.append(jnp.broadcast_to(bias.reshape(1, -1), (8, N)))
    if has_mul:
        in_specs.append(pl.BlockSpec((bm, bn), lambda i, j, k: (i, j)))
        operands.append(mul)

    if out_blocked:
        out_spec = pl.BlockSpec((1, bm, 256), lambda i, j, k: (j, i, 0))
        out_shape = jax.ShapeDtypeStruct((N // 256, M, 256), jnp.float32)
    else:
        out_spec = pl.BlockSpec((bm, bn), lambda i, j, k: (i, j))
        out_shape = jax.ShapeDtypeStruct((M, N), jnp.float32)

    return pl.pallas_call(
        body,
        grid=(M // bm, N // bn, nk),
        in_specs=in_specs,
        out_specs=out_spec,
        out_shape=out_shape,
    )(*operands)


def _pool_mean(r, gid_b, n_graphs, bn=256, bk=256):
    NP, N = r.shape
    bk = min(bk, NP)
    nk = NP // bk
    G = n_graphs

    def body(gid_ref, r_ref, o_ref, cnt_ref):
        k = pl.program_id(1)
        gid = gid_ref[0, 0, :]
        oh = (lax.broadcasted_iota(jnp.int32, (G, bk), 0) ==
              gid[None, :]).astype(jnp.float32)
        acc = jnp.dot(oh.astype(jnp.bfloat16), r_ref[...].astype(jnp.bfloat16),
                      preferred_element_type=jnp.float32)
        cnt = jnp.sum(oh, axis=1, keepdims=True)

        @pl.when(k == 0)
        def _():
            o_ref[...] = jnp.zeros_like(o_ref)
            cnt_ref[...] = jnp.zeros_like(cnt_ref)

        o_ref[...] += acc
        cnt_ref[...] += jnp.broadcast_to(cnt, cnt_ref.shape)

        @pl.when(k == nk - 1)
        def _():
            c = cnt_ref[...][:, 0:1]
            o_ref[...] = o_ref[...] / jnp.maximum(c, 1.0)

    return pl.pallas_call(
        body,
        grid=(N // bn, nk),
        in_specs=[pl.BlockSpec((1, 1, bk), lambda j, k: (k, 0, 0)),
                  pl.BlockSpec((bk, bn), lambda j, k: (k, j))],
        out_specs=pl.BlockSpec((G, bn), lambda j, k: (0, j)),
        out_shape=jax.ShapeDtypeStruct((G, N), jnp.float32),
        scratch_shapes=[pltpu.VMEM((G, 128), jnp.float32)],
    )(gid_b, r)


def _sc_edge(hm2, emsg2, srcs, perm, dsts, bounds, NP, EP, mode, CH=128):
    NPL = hm2.shape[0] // NP
    W = hm2.shape[1]
    n_tiles = _NC * _NS
    slab = NP // n_tiles

    mesh = plsc.VectorSubcoreMesh(core_axis_name="c", subcore_axis_name="s",
                                  num_cores=_NC, num_subcores=_NS)

    def body(hm_ref, emsg_ref, src_ref, perm_ref, dst_ref, bounds_ref,
             out_ref, acc, gbuf, ebuf, sidx, pidx, gidx, eidx, dstv,
             bnd, sem, sem2):
        c_id = lax.axis_index("c")
        s_id = lax.axis_index("s")
        wid = c_id * _NS + s_id
        base_node = wid * slab

        pltpu.sync_copy(bounds_ref, bnd)
        bv = bnd[pl.ds(wid, 16)]
        lo = bv[0]
        hi = bv[1]
        abase = (lo // 8) * 8
        nch = lax.div(hi - abase + CH - 1, CH)

        for p in range(NPL):
            def zr(rr, _):
                for j in range(W // 16):
                    acc[rr, pl.ds(j * 16, 16)] = jnp.zeros((16,), jnp.float32)
                return 0
            lax.fori_loop(0, slab, zr, 0)

            def chunk(ch, _):
                cbase = abase + ch * CH
                pltpu.sync_copy(src_ref.at[pl.ds(cbase, CH)], sidx)
                pltpu.sync_copy(perm_ref.at[pl.ds(cbase, CH)], pidx)
                pltpu.sync_copy(dst_ref.at[pl.ds(cbase, CH)],
                                dstv.at[pl.ds(0, CH)])
                for i in range(CH // 16):
                    sl = pl.ds(i * 16, 16)
                    gidx[sl] = sidx[sl] * NPL + p
                    eidx[sl] = pidx[sl] + p * EP
                cp1 = pltpu.async_copy(hm_ref.at[gidx], gbuf, sem)
                cp2 = pltpu.async_copy(emsg_ref.at[eidx], ebuf, sem2)
                cp1.wait()
                cp2.wait()

                r0 = jnp.maximum(lo - cbase, 0)
                r1 = jnp.minimum(hi - cbase, CH)

                def edge(rr, _):
                    dl = dstv[pl.ds(rr, 16)][0] - base_node
                    for j in range(W // 16):
                        sl = pl.ds(j * 16, 16)
                        g = gbuf[rr, sl]
                        e = ebuf[rr, sl]
                        if mode == 'relu_add':
                            m = jnp.maximum(g + e, 0.0)
                        else:
                            m = g * e
                        acc[dl, sl] = acc[dl, sl] + m
                    return 0
                lax.fori_loop(r0, r1, edge, 0)
                return 0
            lax.fori_loop(0, nch, chunk, 0)

            pltpu.sync_copy(acc, out_ref.at[p, pl.ds(base_node, slab)])

    return pl.kernel(
        body,
        out_type=jax.ShapeDtypeStruct((NPL, NP, W), jnp.float32),
        mesh=mesh,
        scratch_types=[
            pltpu.VMEM((slab, W), jnp.float32),
            pltpu.VMEM((CH, W), jnp.float32),
            pltpu.VMEM((CH, W), jnp.float32),
            pltpu.VMEM((CH,), jnp.int32),
            pltpu.VMEM((CH,), jnp.int32),
            pltpu.VMEM((CH,), jnp.int32),
            pltpu.VMEM((CH,), jnp.int32),
            pltpu.VMEM((CH + 16,), jnp.int32),
            pltpu.VMEM((48,), jnp.int32),
            pltpu.SemaphoreType.DMA,
            pltpu.SemaphoreType.DMA,
        ],
    )(hm2, emsg2, srcs, perm, dsts, bounds)


def kernel(x, edge_index, edge_attr, graph_ids, W_in, b_in, W_msg, b_msg,
           W_node, b_node, W_pn, W_pe, W_ps, W_r1, b_r1, W_r2, b_r2):
    N, DA = x.shape
    E, DB = edge_attr.shape
    D = W_in.shape[1]
    G = _N_GRAPHS
    n_layers = 4
    NPL = D // _W

    NP = _ceil_to(N, 256)
    EP = _ceil_to(E, 256)
    KA = _ceil_to(DA, 128)
    KB = _ceil_to(DB, 128)
    CH = min(128, EP)

    x_p = jnp.pad(x, ((0, NP - N), (0, KA - DA)))
    W_in_p = jnp.pad(W_in, ((0, KA - DA), (0, 0)))
    ea_p = jnp.pad(edge_attr, ((0, EP - E), (0, KB - DB)))
    Wm1 = W_msg[:D]
    Wm2_p = jnp.pad(W_msg[D:], ((0, KB - DB), (0, 0)))
    W_pe_p = jnp.pad(W_pe, ((0, KB - DB), (0, 0)))
    Wn1 = W_node[:D]
    Wn2 = W_node[D:]

    src = jnp.pad(edge_index[0], (0, EP - E), constant_values=0)
    dst = jnp.pad(edge_index[1], (0, EP - E), constant_values=NP - 1)
    gid_b = jnp.pad(graph_ids, (0, NP - N),
                    constant_values=G).reshape(NP // 256, 1, 256)

    n_tiles = _NC * _NS
    slab_n = NP // n_tiles
    perm0 = jnp.argsort(dst).astype(jnp.int32)
    dst_s = dst[perm0]
    src_s = src[perm0]
    bounds = jnp.searchsorted(
        dst_s, jnp.arange(n_tiles + 1, dtype=jnp.int32) * slab_n
    ).astype(jnp.int32)
    bounds = jnp.pad(bounds, (0, 48 - n_tiles - 1), constant_values=EP)
    srcs = jnp.pad(src_s, (0, CH))
    perm_p = jnp.pad(perm0, (0, CH))
    dsts = jnp.pad(dst_s, (0, CH), constant_values=NP - 1)

    h = _dense_mm([(x_p, W_in_p, False, None)], bias=b_in, act=_relu, bk=KA)
    emsg_b = _dense_mm([(ea_p, Wm2_p, False, None)], bias=b_msg, bk=KB,
                       out_blocked=True)
    he_b = _dense_mm([(ea_p, W_pe_p, False, None)], bk=KB, out_blocked=True)

    emsg2 = emsg_b.reshape(-1, _W)
    he2 = he_b.reshape(-1, _W)
    for _ in range(n_layers):
        hm = _dense_mm([(h, Wm1, False, None)])
        agg_b = _sc_edge(hm.reshape(-1, _W), emsg2, srcs, perm_p, dsts,
                         bounds, NP, EP, 'relu_add', CH=CH)
        h = _dense_mm([(h, Wn1, False, None), (agg_b, Wn2, True, None)],
                      bias=b_node, act=_relu)

    hv = _dense_mm([(h, W_pn, False, None)])
    hs = _dense_mm([(h, W_ps, False, None)])
    aggp_b = _sc_edge(hv.reshape(-1, _W), he2, srcs, perm_p, dsts,
                      bounds, NP, EP, 'mul', CH=CH)
    r = _dense_mm([(aggp_b, W_r1, True, hs)], bias=b_r1, act=_leaky)
    pooled = _pool_mean(r, gid_b, G)
    out = _dense_mm([(pooled, W_r2, False, None)], bias=b_r2, bm=128)
    return out


# speedup vs baseline: 7.0350x; 1.1855x over previous
import jax
import jax.numpy as jnp
from jax import lax
from jax.experimental import pallas as pl
from jax.experimental.pallas import tpu as pltpu
from jax.experimental.pallas import tpu_sc as plsc

_N_GRAPHS = 128
_NC = 2
_NS = 16
_W = 256


def _ceil_to(a, b):
    return (a + b - 1) // b * b


def _relu(v):
    return jnp.maximum(v, 0.0)


def _leaky(v):
    return jnp.where(v > 0, v, 0.01 * v)


def _dense_mm(pairs, bias=None, add=None, mul=None, act=None, bm=512, bn=512,
              bk=512, out_blocked=False, out_dtype=jnp.float32):
    a0 = pairs[0][0]
    M = a0.shape[1] if pairs[0][2] else a0.shape[0]
    K = a0.shape[0] * 256 if pairs[0][2] else a0.shape[1]
    N = pairs[0][1].shape[1]
    bm = min(bm, M)
    bk = min(bk, K)
    if out_blocked:
        bn = 256
    bn = min(bn, N)
    nk = K // bk
    nsub = bk // 256
    has_bias = bias is not None
    has_add = add is not None
    has_mul = mul is not None

    def _bf(v):
        return v if v.dtype == jnp.bfloat16 else v.astype(jnp.bfloat16)

    def body(*refs):
        acc_ref = refs[-1]
        o_ref = refs[-2]
        k = pl.program_id(2)
        acc = jnp.zeros((bm, bn), jnp.float32)
        idx = 0
        for (_, _, blocked, amul) in pairs:
            a_ref = refs[idx]
            idx += 1
            b_ref = refs[idx]
            idx += 1
            if amul is not None:
                am_ref = refs[idx]
                idx += 1
            if blocked:
                b_all = b_ref[...]
                for q in range(nsub):
                    a = a_ref[q]
                    if amul is not None:
                        a = a * am_ref[..., q * 256:(q + 1) * 256]
                    acc = acc + jnp.dot(
                        _bf(a), _bf(b_all[q * 256:(q + 1) * 256, :]),
                        preferred_element_type=jnp.float32)
            else:
                a = a_ref[...]
                if amul is not None:
                    a = a * am_ref[...]
                acc = acc + jnp.dot(_bf(a), _bf(b_ref[...]),
                                    preferred_element_type=jnp.float32)
        n_in = idx

        @pl.when(k == 0)
        def _():
            acc_ref[...] = jnp.zeros((bm, bn), jnp.float32)

        acc_ref[...] += acc

        @pl.when(k == nk - 1)
        def _():
            v = acc_ref[...]
            i_extra = n_in
            if has_add:
                v = v + refs[i_extra][...]
                i_extra += 1
            if has_bias:
                v = v + refs[i_extra][0:1, :]
                i_extra += 1
            if act is not None:
                v = act(v)
            if has_mul:
                v = v * refs[i_extra][...]
            v = v.astype(out_dtype)
            if out_blocked:
                o_ref[0] = v
            else:
                o_ref[...] = v

    in_specs = []
    operands = []
    for (a, b, blocked, amul) in pairs:
        if blocked:
            in_specs.append(
                pl.BlockSpec((nsub, bm, 256), lambda i, j, k: (k, i, 0)))
        else:
            in_specs.append(pl.BlockSpec((bm, bk), lambda i, j, k: (i, k)))
        in_specs.append(pl.BlockSpec((bk, bn), lambda i, j, k: (k, j)))
        operands += [a, b]
        if amul is not None:
            in_specs.append(pl.BlockSpec((bm, bk), lambda i, j, k: (i, k)))
            operands.append(amul)
    if has_add:
        in_specs.append(pl.BlockSpec((bm, bn), lambda i, j, k: (i, j)))
        operands.append(add)
    if has_bias:
        in_specs.append(pl.BlockSpec((8, bn), lambda i, j, k: (0, j)))
        operands.append(jnp.broadcast_to(bias.reshape(1, -1), (8, N)))
    if has_mul:
        in_specs.append(pl.BlockSpec((bm, bn), lambda i, j, k: (i, j)))
        operands.append(mul)

    if out_blocked:
        out_spec = pl.BlockSpec((1, bm, 256), lambda i, j, k: (j, i, 0))
        out_shape = jax.ShapeDtypeStruct((N // 256, M, 256), out_dtype)
    else:
        out_spec = pl.BlockSpec((bm, bn), lambda i, j, k: (i, j))
        out_shape = jax.ShapeDtypeStruct((M, N), out_dtype)

    return pl.pallas_call(
        body,
        grid=(M // bm, N // bn, nk),
        in_specs=in_specs,
        out_specs=out_spec,
        out_shape=out_shape,
        scratch_shapes=[pltpu.VMEM((bm, bn), jnp.float32)],
    )(*operands)


def _pool_mean(r, gid_b, n_graphs, bn=256, bk=256):
    NP, N = r.shape
    bk = min(bk, NP)
    nk = NP // bk
    G = n_graphs

    def body(gid_ref, r_ref, o_ref, cnt_ref):
        k = pl.program_id(1)
        gid = gid_ref[0, 0, :]
        oh = (lax.broadcasted_iota(jnp.int32, (G, bk), 0) ==
              gid[None, :]).astype(jnp.float32)
        acc = jnp.dot(oh.astype(jnp.bfloat16), r_ref[...].astype(jnp.bfloat16),
                      preferred_element_type=jnp.float32)
        cnt = jnp.sum(oh, axis=1, keepdims=True)

        @pl.when(k == 0)
        def _():
            o_ref[...] = jnp.zeros_like(o_ref)
            cnt_ref[...] = jnp.zeros_like(cnt_ref)

        o_ref[...] += acc
        cnt_ref[...] += jnp.broadcast_to(cnt, cnt_ref.shape)

        @pl.when(k == nk - 1)
        def _():
            c = cnt_ref[...][:, 0:1]
            o_ref[...] = o_ref[...] / jnp.maximum(c, 1.0)

    return pl.pallas_call(
        body,
        grid=(N // bn, nk),
        in_specs=[pl.BlockSpec((1, 1, bk), lambda j, k: (k, 0, 0)),
                  pl.BlockSpec((bk, bn), lambda j, k: (k, j))],
        out_specs=pl.BlockSpec((G, bn), lambda j, k: (0, j)),
        out_shape=jax.ShapeDtypeStruct((G, N), jnp.float32),
        scratch_shapes=[pltpu.VMEM((G, 128), jnp.float32)],
    )(gid_b, r)


def _sc_edge(hm2, emsg2, srcs, perm, dsts, bounds, NP, EP, mode, CH=128):
    NPL = hm2.shape[0] // NP
    W = hm2.shape[1]
    n_tiles = _NC * _NS
    slab = NP // n_tiles

    mesh = plsc.VectorSubcoreMesh(core_axis_name="c", subcore_axis_name="s",
                                  num_cores=_NC, num_subcores=_NS)

    def body(hm_ref, emsg_ref, src_ref, perm_ref, dst_ref, bounds_ref,
             out_ref, acc, gbuf, ebuf, sidx, pidx, gidx, eidx, dstv,
             bnd, sem, sem2):
        c_id = lax.axis_index("c")
        s_id = lax.axis_index("s")
        wid = c_id * _NS + s_id
        base_node = wid * slab

        pltpu.sync_copy(bounds_ref, bnd)
        bv = bnd[pl.ds(wid, 16)]
        lo = bv[0]
        hi = bv[1]
        abase = (lo // 8) * 8
        nch = lax.div(hi - abase + CH - 1, CH)

        for p in range(NPL):
            def zr(rr, _):
                for j in range(W // 16):
                    acc[rr, pl.ds(j * 16, 16)] = jnp.zeros((16,), jnp.float32)
                return 0
            lax.fori_loop(0, slab, zr, 0)

            def chunk(ch, _):
                cbase = abase + ch * CH
                pltpu.sync_copy(src_ref.at[pl.ds(cbase, CH)], sidx)
                pltpu.sync_copy(perm_ref.at[pl.ds(cbase, CH)], pidx)
                pltpu.sync_copy(dst_ref.at[pl.ds(cbase, CH)],
                                dstv.at[pl.ds(0, CH)])
                for i in range(CH // 16):
                    sl = pl.ds(i * 16, 16)
                    gidx[sl] = sidx[sl] * NPL + p
                    eidx[sl] = pidx[sl] + p * EP
                cp1 = pltpu.async_copy(hm_ref.at[gidx], gbuf, sem)
                cp2 = pltpu.async_copy(emsg_ref.at[eidx], ebuf, sem2)
                cp1.wait()
                cp2.wait()

                r0 = jnp.maximum(lo - cbase, 0)
                r1 = jnp.minimum(hi - cbase, CH)

                def edge(rr, _):
                    dl = dstv[pl.ds(rr, 16)][0] - base_node
                    for j in range(W // 16):
                        sl = pl.ds(j * 16, 16)
                        g = gbuf[rr, sl]
                        e = ebuf[rr, sl]
                        if mode == 'relu_add':
                            m = jnp.maximum(g + e, 0.0)
                        else:
                            m = g * e
                        acc[dl, sl] = acc[dl, sl] + m
                    return 0
                lax.fori_loop(r0, r1, edge, 0)
                return 0
            lax.fori_loop(0, nch, chunk, 0)

            pltpu.sync_copy(acc, out_ref.at[p, pl.ds(base_node, slab)])

    return pl.kernel(
        body,
        out_type=jax.ShapeDtypeStruct((NPL, NP, W), jnp.float32),
        mesh=mesh,
        scratch_types=[
            pltpu.VMEM((slab, W), jnp.float32),
            pltpu.VMEM((CH, W), jnp.float32),
            pltpu.VMEM((CH, W), jnp.float32),
            pltpu.VMEM((CH,), jnp.int32),
            pltpu.VMEM((CH,), jnp.int32),
            pltpu.VMEM((CH,), jnp.int32),
            pltpu.VMEM((CH,), jnp.int32),
            pltpu.VMEM((CH + 16,), jnp.int32),
            pltpu.VMEM((48,), jnp.int32),
            pltpu.SemaphoreType.DMA,
            pltpu.SemaphoreType.DMA,
        ],
    )(hm2, emsg2, srcs, perm, dsts, bounds)


def kernel(x, edge_index, edge_attr, graph_ids, W_in, b_in, W_msg, b_msg,
           W_node, b_node, W_pn, W_pe, W_ps, W_r1, b_r1, W_r2, b_r2):
    N, DA = x.shape
    E, DB = edge_attr.shape
    D = W_in.shape[1]
    G = _N_GRAPHS
    n_layers = 4
    NPL = D // _W

    NP = _ceil_to(N, 256)
    EP = _ceil_to(E, 256)
    KA = _ceil_to(DA, 128)
    KB = _ceil_to(DB, 128)
    CH = min(128, EP)

    bf = jnp.bfloat16
    x_p = jnp.pad(x, ((0, NP - N), (0, KA - DA))).astype(bf)
    W_in_p = jnp.pad(W_in, ((0, KA - DA), (0, 0))).astype(bf)
    ea_p = jnp.pad(edge_attr, ((0, EP - E), (0, KB - DB))).astype(bf)
    Wm1 = W_msg[:D].astype(bf)
    Wm2_p = jnp.pad(W_msg[D:], ((0, KB - DB), (0, 0))).astype(bf)
    W_pe_p = jnp.pad(W_pe, ((0, KB - DB), (0, 0))).astype(bf)
    Wn1 = W_node[:D].astype(bf)
    Wn2 = W_node[D:].astype(bf)

    src = jnp.pad(edge_index[0], (0, EP - E), constant_values=0)
    dst = jnp.pad(edge_index[1], (0, EP - E), constant_values=NP - 1)
    gid_b = jnp.pad(graph_ids, (0, NP - N),
                    constant_values=G).reshape(NP // 256, 1, 256)

    n_tiles = _NC * _NS
    slab_n = NP // n_tiles
    perm0 = jnp.argsort(dst).astype(jnp.int32)
    dst_s = dst[perm0]
    src_s = src[perm0]
    bounds = jnp.searchsorted(
        dst_s, jnp.arange(n_tiles + 1, dtype=jnp.int32) * slab_n
    ).astype(jnp.int32)
    bounds = jnp.pad(bounds, (0, 48 - n_tiles - 1), constant_values=EP)
    srcs = jnp.pad(src_s, (0, CH))
    perm_p = jnp.pad(perm0, (0, CH))
    dsts = jnp.pad(dst_s, (0, CH), constant_values=NP - 1)

    h = _dense_mm([(x_p, W_in_p, False, None)], bias=b_in, act=_relu, bk=KA,
                  out_dtype=bf)
    emsg_b = _dense_mm([(ea_p, Wm2_p, False, None)], bias=b_msg, bk=KB,
                       out_blocked=True)
    he_b = _dense_mm([(ea_p, W_pe_p, False, None)], bk=KB, out_blocked=True)

    emsg2 = emsg_b.reshape(-1, _W)
    he2 = he_b.reshape(-1, _W)
    for _ in range(n_layers):
        hm = _dense_mm([(h, Wm1, False, None)])
        n1 = _dense_mm([(h, Wn1, False, None)])
        agg_b = _sc_edge(hm.reshape(-1, _W), emsg2, srcs, perm_p, dsts,
                         bounds, NP, EP, 'relu_add', CH=CH)
        h = _dense_mm([(agg_b, Wn2, True, None)], add=n1,
                      bias=b_node, act=_relu, out_dtype=bf)

    hv = _dense_mm([(h, W_pn.astype(bf), False, None)])
    hs = _dense_mm([(h, W_ps.astype(bf), False, None)])
    aggp_b = _sc_edge(hv.reshape(-1, _W), he2, srcs, perm_p, dsts,
                      bounds, NP, EP, 'mul', CH=CH)
    r = _dense_mm([(aggp_b, W_r1.astype(bf), True, hs)], bias=b_r1,
                  act=_leaky, out_dtype=bf)
    pooled = _pool_mean(r, gid_b, G)
    out = _dense_mm([(pooled, W_r2.astype(bf), False, None)], bias=b_r2,
                    bm=128)
    return out


# speedup vs baseline: 7.0713x; 1.0052x over previous
import jax
import jax.numpy as jnp
from jax import lax
from jax.experimental import pallas as pl
from jax.experimental.pallas import tpu as pltpu
from jax.experimental.pallas import tpu_sc as plsc

_N_GRAPHS = 128
_NC = 2
_NS = 16
_W = 256


def _ceil_to(a, b):
    return (a + b - 1) // b * b


def _relu(v):
    return jnp.maximum(v, 0.0)


def _leaky(v):
    return jnp.where(v > 0, v, 0.01 * v)


def _dense_mm(pairs, bias=None, add=None, mul=None, act=None, bm=512, bn=512,
              bk=512, out_blocked=False, out_dtype=jnp.float32):
    a0 = pairs[0][0]
    M = a0.shape[1] if pairs[0][2] else a0.shape[0]
    K = a0.shape[0] * 256 if pairs[0][2] else a0.shape[1]
    N = pairs[0][1].shape[1]
    bm = min(bm, M)
    bk = min(bk, K)
    if out_blocked:
        bn = 256
    bn = min(bn, N)
    nk = K // bk
    nsub = bk // 256
    has_bias = bias is not None
    has_add = add is not None
    has_mul = mul is not None

    def _bf(v):
        return v if v.dtype == jnp.bfloat16 else v.astype(jnp.bfloat16)

    def body(*refs):
        acc_ref = refs[-1]
        o_ref = refs[-2]
        k = pl.program_id(2)
        acc = jnp.zeros((bm, bn), jnp.float32)
        idx = 0
        for (_, _, blocked, amul) in pairs:
            a_ref = refs[idx]
            idx += 1
            b_ref = refs[idx]
            idx += 1
            if amul is not None:
                am_ref = refs[idx]
                idx += 1
            if blocked:
                b_all = b_ref[...]
                for q in range(nsub):
                    a = a_ref[q]
                    if amul is not None:
                        a = a * am_ref[..., q * 256:(q + 1) * 256]
                    acc = acc + jnp.dot(
                        _bf(a), _bf(b_all[q * 256:(q + 1) * 256, :]),
                        preferred_element_type=jnp.float32)
            else:
                a = a_ref[...]
                if amul is not None:
                    a = a * am_ref[...]
                acc = acc + jnp.dot(_bf(a), _bf(b_ref[...]),
                                    preferred_element_type=jnp.float32)
        n_in = idx

        @pl.when(k == 0)
        def _():
            acc_ref[...] = jnp.zeros((bm, bn), jnp.float32)

        acc_ref[...] += acc

        @pl.when(k == nk - 1)
        def _():
            v = acc_ref[...]
            i_extra = n_in
            if has_add:
                v = v + refs[i_extra][...]
                i_extra += 1
            if has_bias:
                v = v + refs[i_extra][0:1, :]
                i_extra += 1
            if act is not None:
                v = act(v)
            if has_mul:
                v = v * refs[i_extra][...]
            v = v.astype(out_dtype)
            if out_blocked:
                o_ref[0] = v
            else:
                o_ref[...] = v

    in_specs = []
    operands = []
    for (a, b, blocked, amul) in pairs:
        if blocked:
            in_specs.append(
                pl.BlockSpec((nsub, bm, 256), lambda i, j, k: (k, i, 0)))
        else:
            in_specs.append(pl.BlockSpec((bm, bk), lambda i, j, k: (i, k)))
        in_specs.append(pl.BlockSpec((bk, bn), lambda i, j, k: (k, j)))
        operands += [a, b]
        if amul is not None:
            in_specs.append(pl.BlockSpec((bm, bk), lambda i, j, k: (i, k)))
            operands.append(amul)
    if has_add:
        in_specs.append(pl.BlockSpec((bm, bn), lambda i, j, k: (i, j)))
        operands.append(add)
    if has_bias:
        in_specs.append(pl.BlockSpec((8, bn), lambda i, j, k: (0, j)))
        operands.append(jnp.broadcast_to(bias.reshape(1, -1), (8, N)))
    if has_mul:
        in_specs.append(pl.BlockSpec((bm, bn), lambda i, j, k: (i, j)))
        operands.append(mul)

    if out_blocked:
        out_spec = pl.BlockSpec((1, bm, 256), lambda i, j, k: (j, i, 0))
        out_shape = jax.ShapeDtypeStruct((N // 256, M, 256), out_dtype)
    else:
        out_spec = pl.BlockSpec((bm, bn), lambda i, j, k: (i, j))
        out_shape = jax.ShapeDtypeStruct((M, N), out_dtype)

    return pl.pallas_call(
        body,
        grid=(M // bm, N // bn, nk),
        in_specs=in_specs,
        out_specs=out_spec,
        out_shape=out_shape,
        scratch_shapes=[pltpu.VMEM((bm, bn), jnp.float32)],
    )(*operands)


def _pool_mean(r, gid_b, n_graphs, bn=256, bk=256):
    NP, N = r.shape
    bk = min(bk, NP)
    nk = NP // bk
    G = n_graphs

    def body(gid_ref, r_ref, o_ref, cnt_ref):
        k = pl.program_id(1)
        gid = gid_ref[0, 0, :]
        oh = (lax.broadcasted_iota(jnp.int32, (G, bk), 0) ==
              gid[None, :]).astype(jnp.float32)
        acc = jnp.dot(oh.astype(jnp.bfloat16), r_ref[...].astype(jnp.bfloat16),
                      preferred_element_type=jnp.float32)
        cnt = jnp.sum(oh, axis=1, keepdims=True)

        @pl.when(k == 0)
        def _():
            o_ref[...] = jnp.zeros_like(o_ref)
            cnt_ref[...] = jnp.zeros_like(cnt_ref)

        o_ref[...] += acc
        cnt_ref[...] += jnp.broadcast_to(cnt, cnt_ref.shape)

        @pl.when(k == nk - 1)
        def _():
            c = cnt_ref[...][:, 0:1]
            o_ref[...] = o_ref[...] / jnp.maximum(c, 1.0)

    return pl.pallas_call(
        body,
        grid=(N // bn, nk),
        in_specs=[pl.BlockSpec((1, 1, bk), lambda j, k: (k, 0, 0)),
                  pl.BlockSpec((bk, bn), lambda j, k: (k, j))],
        out_specs=pl.BlockSpec((G, bn), lambda j, k: (0, j)),
        out_shape=jax.ShapeDtypeStruct((G, N), jnp.float32),
        scratch_shapes=[pltpu.VMEM((G, 128), jnp.float32)],
    )(gid_b, r)


def _sc_edge(hm2, emsg2, srcs, eids, dsts, bounds, NP, EP, mode, CH=128):
    NPL = hm2.shape[0] // NP
    W = hm2.shape[1]
    n_tiles = _NC * _NS
    slab = NP // n_tiles

    mesh = plsc.VectorSubcoreMesh(core_axis_name="c", subcore_axis_name="s",
                                  num_cores=_NC, num_subcores=_NS)

    def body(hm_ref, emsg_ref, src_ref, eid_ref, dst_ref, bounds_ref,
             out_ref, acc, gbuf, ebuf, sidx, pidx, gidx, eidx, dstv,
             bnd, sem, sem2):
        c_id = lax.axis_index("c")
        s_id = lax.axis_index("s")
        wid = c_id * _NS + s_id
        base_node = wid * slab

        pltpu.sync_copy(bounds_ref, bnd)
        bv = bnd[pl.ds(wid, 16)]
        lo = bv[0]
        hi = bv[1]
        abase = (lo // 8) * 8
        nch = lax.div(hi - abase + CH - 1, CH)

        for p in range(NPL):
            def zr(rr, _):
                for j in range(W // 16):
                    acc[rr, pl.ds(j * 16, 16)] = jnp.zeros((16,), jnp.float32)
                return 0
            lax.fori_loop(0, slab, zr, 0)

            def chunk(ch, _):
                cbase = abase + ch * CH
                pltpu.sync_copy(src_ref.at[pl.ds(cbase, CH)], sidx)
                pltpu.sync_copy(eid_ref.at[pl.ds(cbase, CH)], pidx)
                pltpu.sync_copy(dst_ref.at[pl.ds(cbase, CH)],
                                dstv.at[pl.ds(0, CH)])
                for i in range(CH // 16):
                    sl = pl.ds(i * 16, 16)
                    gidx[sl] = sidx[sl] * NPL + p
                    eidx[sl] = pidx[sl] + p * EP
                cp1 = pltpu.async_copy(hm_ref.at[gidx], gbuf, sem)
                cp2 = pltpu.async_copy(emsg_ref.at[eidx], ebuf, sem2)
                cp1.wait()
                cp2.wait()

                r0 = jnp.maximum(lo - cbase, 0)
                r1 = jnp.minimum(hi - cbase, CH)

                def edge(rr, _):
                    dl = dstv[pl.ds(rr, 16)][0] - base_node
                    for j in range(W // 16):
                        sl = pl.ds(j * 16, 16)
                        g = gbuf[rr, sl]
                        e = ebuf[rr, sl]
                        if mode == 'relu_add':
                            m = jnp.maximum(g + e, 0.0)
                        else:
                            m = g * e
                        acc[dl, sl] = acc[dl, sl] + m
                    return 0
                lax.fori_loop(r0, r1, edge, 0)
                return 0
            lax.fori_loop(0, nch, chunk, 0)

            pltpu.sync_copy(acc, out_ref.at[p, pl.ds(base_node, slab)])

    return pl.kernel(
        body,
        out_type=jax.ShapeDtypeStruct((NPL, NP, W), jnp.float32),
        mesh=mesh,
        scratch_types=[
            pltpu.VMEM((slab, W), jnp.float32),
            pltpu.VMEM((CH, W), jnp.float32),
            pltpu.VMEM((CH, W), jnp.float32),
            pltpu.VMEM((CH,), jnp.int32),
            pltpu.VMEM((CH,), jnp.int32),
            pltpu.VMEM((CH,), jnp.int32),
            pltpu.VMEM((CH,), jnp.int32),
            pltpu.VMEM((CH + 16,), jnp.int32),
            pltpu.VMEM((48,), jnp.int32),
            pltpu.SemaphoreType.DMA,
            pltpu.SemaphoreType.DMA,
        ],
    )(hm2, emsg2, srcs, eids, dsts, bounds)


def kernel(x, edge_index, edge_attr, graph_ids, W_in, b_in, W_msg, b_msg,
           W_node, b_node, W_pn, W_pe, W_ps, W_r1, b_r1, W_r2, b_r2):
    N, DA = x.shape
    E, DB = edge_attr.shape
    D = W_in.shape[1]
    G = _N_GRAPHS
    n_layers = 4
    NPL = D // _W

    NP = _ceil_to(N, 256)
    EP = _ceil_to(E, 256)
    KA = _ceil_to(DA, 128)
    KB = _ceil_to(DB, 128)
    CH = min(128, EP)

    bf = jnp.bfloat16
    x_p = jnp.pad(x, ((0, NP - N), (0, KA - DA))).astype(bf)
    W_in_p = jnp.pad(W_in, ((0, KA - DA), (0, 0))).astype(bf)
    ea_p = jnp.pad(edge_attr, ((0, EP - E), (0, KB - DB))).astype(bf)
    Wm1 = W_msg[:D].astype(bf)
    Wm2_p = jnp.pad(W_msg[D:], ((0, KB - DB), (0, 0))).astype(bf)
    W_pe_p = jnp.pad(W_pe, ((0, KB - DB), (0, 0))).astype(bf)
    Wn1 = W_node[:D].astype(bf)
    Wn2 = W_node[D:].astype(bf)

    src = jnp.pad(edge_index[0], (0, EP - E), constant_values=0)
    dst = jnp.pad(edge_index[1], (0, EP - E), constant_values=NP - 1)
    gid_b = jnp.pad(graph_ids, (0, NP - N),
                    constant_values=G).reshape(NP // 256, 1, 256)

    n_tiles = _NC * _NS
    slab_n = NP // n_tiles
    perm0 = jnp.argsort(dst).astype(jnp.int32)
    dst_s = dst[perm0]
    src_s = src[perm0]
    bounds = jnp.searchsorted(
        dst_s, jnp.arange(n_tiles + 1, dtype=jnp.int32) * slab_n
    ).astype(jnp.int32)
    bounds = jnp.pad(bounds, (0, 48 - n_tiles - 1), constant_values=EP)
    srcs = jnp.pad(src_s, (0, CH))
    dsts = jnp.pad(dst_s, (0, CH), constant_values=NP - 1)
    eids = jnp.arange(EP + CH, dtype=jnp.int32)
    ea_s = ea_p[perm0]

    h = _dense_mm([(x_p, W_in_p, False, None)], bias=b_in, act=_relu, bk=KA,
                  out_dtype=bf)
    emsg_b = _dense_mm([(ea_s, Wm2_p, False, None)], bias=b_msg, bk=KB,
                       out_blocked=True)
    he_b = _dense_mm([(ea_s, W_pe_p, False, None)], bk=KB, out_blocked=True)

    emsg2 = emsg_b.reshape(-1, _W)
    he2 = he_b.reshape(-1, _W)
    for _ in range(n_layers):
        hm = _dense_mm([(h, Wm1, False, None)])
        n1 = _dense_mm([(h, Wn1, False, None)])
        agg_b = _sc_edge(hm.reshape(-1, _W), emsg2, srcs, eids, dsts,
                         bounds, NP, EP, 'relu_add', CH=CH)
        h = _dense_mm([(agg_b, Wn2, True, None)], add=n1,
                      bias=b_node, act=_relu, out_dtype=bf)

    hv = _dense_mm([(h, W_pn.astype(bf), False, None)])
    hs = _dense_mm([(h, W_ps.astype(bf), False, None)])
    aggp_b = _sc_edge(hv.reshape(-1, _W), he2, srcs, eids, dsts,
                      bounds, NP, EP, 'mul', CH=CH)
    r = _dense_mm([(aggp_b, W_r1.astype(bf), True, hs)], bias=b_r1,
                  act=_leaky, out_dtype=bf)
    pooled = _pool_mean(r, gid_b, G)
    out = _dense_mm([(pooled, W_r2.astype(bf), False, None)], bias=b_r2,
                    bm=128)
    return out


# speedup vs baseline: 8.6474x; 1.2229x over previous
import jax
import jax.numpy as jnp
from jax import lax
from jax.experimental import pallas as pl
from jax.experimental.pallas import tpu as pltpu
from jax.experimental.pallas import tpu_sc as plsc

_N_GRAPHS = 128
_NC = 2
_NS = 16
_W = 256


def _ceil_to(a, b):
    return (a + b - 1) // b * b


def _relu(v):
    return jnp.maximum(v, 0.0)


def _leaky(v):
    return jnp.where(v > 0, v, 0.01 * v)


def _dense_mm(pairs, bias=None, add=None, mul=None, act=None, bm=1024,
              bn=1024, bk=512, out_blocked=False, out_dtype=jnp.float32):
    a0 = pairs[0][0]
    M = a0.shape[1] if pairs[0][2] else a0.shape[0]
    K = a0.shape[0] * 256 if pairs[0][2] else a0.shape[1]
    N = pairs[0][1].shape[1]
    bm = min(bm, M)
    bk = min(bk, K)
    if out_blocked:
        bn = 256
    bn = min(bn, N)
    nk = K // bk
    nsub = bk // 256
    has_bias = bias is not None
    has_add = add is not None
    has_mul = mul is not None

    def _bf(v):
        return v if v.dtype == jnp.bfloat16 else v.astype(jnp.bfloat16)

    def body(*refs):
        acc_ref = refs[-1]
        o_ref = refs[-2]
        k = pl.program_id(2)
        acc = jnp.zeros((bm, bn), jnp.float32)
        idx = 0
        for (_, _, blocked, amul) in pairs:
            a_ref = refs[idx]
            idx += 1
            b_ref = refs[idx]
            idx += 1
            if amul is not None:
                am_ref = refs[idx]
                idx += 1
            if blocked:
                b_all = b_ref[...]
                for q in range(nsub):
                    a = a_ref[q]
                    if amul is not None:
                        a = a * am_ref[..., q * 256:(q + 1) * 256]
                    acc = acc + jnp.dot(
                        _bf(a), _bf(b_all[q * 256:(q + 1) * 256, :]),
                        preferred_element_type=jnp.float32)
            else:
                a = a_ref[...]
                if amul is not None:
                    a = a * am_ref[...]
                acc = acc + jnp.dot(_bf(a), _bf(b_ref[...]),
                                    preferred_element_type=jnp.float32)
        n_in = idx

        @pl.when(k == 0)
        def _():
            acc_ref[...] = jnp.zeros((bm, bn), jnp.float32)

        acc_ref[...] += acc

        @pl.when(k == nk - 1)
        def _():
            v = acc_ref[...]
            i_extra = n_in
            if has_add:
                v = v + refs[i_extra][...]
                i_extra += 1
            if has_bias:
                v = v + refs[i_extra][0:1, :]
                i_extra += 1
            if act is not None:
                v = act(v)
            if has_mul:
                v = v * refs[i_extra][...]
            v = v.astype(out_dtype)
            if out_blocked:
                o_ref[0] = v
            else:
                o_ref[...] = v

    in_specs = []
    operands = []
    for (a, b, blocked, amul) in pairs:
        if blocked:
            in_specs.append(
                pl.BlockSpec((nsub, bm, 256), lambda i, j, k: (k, i, 0)))
        else:
            in_specs.append(pl.BlockSpec((bm, bk), lambda i, j, k: (i, k)))
        in_specs.append(pl.BlockSpec((bk, bn), lambda i, j, k: (k, j)))
        operands += [a, b]
        if amul is not None:
            in_specs.append(pl.BlockSpec((bm, bk), lambda i, j, k: (i, k)))
            operands.append(amul)
    if has_add:
        in_specs.append(pl.BlockSpec((bm, bn), lambda i, j, k: (i, j)))
        operands.append(add)
    if has_bias:
        in_specs.append(pl.BlockSpec((8, bn), lambda i, j, k: (0, j)))
        operands.append(jnp.broadcast_to(bias.reshape(1, -1), (8, N)))
    if has_mul:
        in_specs.append(pl.BlockSpec((bm, bn), lambda i, j, k: (i, j)))
        operands.append(mul)

    if out_blocked:
        out_spec = pl.BlockSpec((1, bm, 256), lambda i, j, k: (j, i, 0))
        out_shape = jax.ShapeDtypeStruct((N // 256, M, 256), out_dtype)
    else:
        out_spec = pl.BlockSpec((bm, bn), lambda i, j, k: (i, j))
        out_shape = jax.ShapeDtypeStruct((M, N), out_dtype)

    return pl.pallas_call(
        body,
        grid=(M // bm, N // bn, nk),
        in_specs=in_specs,
        out_specs=out_spec,
        out_shape=out_shape,
        scratch_shapes=[pltpu.VMEM((bm, bn), jnp.float32)],
    )(*operands)


def _pool_mean(r, gid_b, n_graphs, bn=256, bk=256):
    NP, N = r.shape
    bk = min(bk, NP)
    nk = NP // bk
    G = n_graphs

    def body(gid_ref, r_ref, o_ref, cnt_ref):
        k = pl.program_id(1)
        gid = gid_ref[0, 0, :]
        oh = (lax.broadcasted_iota(jnp.int32, (G, bk), 0) ==
              gid[None, :]).astype(jnp.float32)
        acc = jnp.dot(oh.astype(jnp.bfloat16), r_ref[...].astype(jnp.bfloat16),
                      preferred_element_type=jnp.float32)
        cnt = jnp.sum(oh, axis=1, keepdims=True)

        @pl.when(k == 0)
        def _():
            o_ref[...] = jnp.zeros_like(o_ref)
            cnt_ref[...] = jnp.zeros_like(cnt_ref)

        o_ref[...] += acc
        cnt_ref[...] += jnp.broadcast_to(cnt, cnt_ref.shape)

        @pl.when(k == nk - 1)
        def _():
            c = cnt_ref[...][:, 0:1]
            o_ref[...] = o_ref[...] / jnp.maximum(c, 1.0)

    return pl.pallas_call(
        body,
        grid=(N // bn, nk),
        in_specs=[pl.BlockSpec((1, 1, bk), lambda j, k: (k, 0, 0)),
                  pl.BlockSpec((bk, bn), lambda j, k: (k, j))],
        out_specs=pl.BlockSpec((G, bn), lambda j, k: (0, j)),
        out_shape=jax.ShapeDtypeStruct((G, N), jnp.float32),
        scratch_shapes=[pltpu.VMEM((G, 128), jnp.float32)],
    )(gid_b, r)


def _sc_edge(hm2, emsg2, srcs, eids, dsts, bounds, NP, EP, mode, CH=128):
    NPL = hm2.shape[0] // NP
    W = hm2.shape[1]
    n_tiles = _NC * _NS
    slab = NP // n_tiles

    mesh = plsc.VectorSubcoreMesh(core_axis_name="c", subcore_axis_name="s",
                                  num_cores=_NC, num_subcores=_NS)

    def body(hm_ref, emsg_ref, src_ref, eid_ref, dst_ref, bounds_ref,
             out_ref, acc, gbuf, ebuf, sidx, pidx, gidx, eidx, dstv,
             bnd, sem, sem2):
        c_id = lax.axis_index("c")
        s_id = lax.axis_index("s")
        wid = c_id * _NS + s_id
        base_node = wid * slab

        pltpu.sync_copy(bounds_ref, bnd)
        bv = bnd[pl.ds(wid, 16)]
        lo = bv[0]
        hi = bv[1]
        abase = (lo // 8) * 8
        nch = lax.div(hi - abase + CH - 1, CH)

        for p in range(NPL):
            def zr(rr, _):
                for j in range(W // 16):
                    acc[rr, pl.ds(j * 16, 16)] = jnp.zeros((16,), jnp.float32)
                return 0
            lax.fori_loop(0, slab, zr, 0)

            def chunk(ch, _):
                cbase = abase + ch * CH
                pltpu.sync_copy(src_ref.at[pl.ds(cbase, CH)], sidx)
                pltpu.sync_copy(eid_ref.at[pl.ds(cbase, CH)], pidx)
                pltpu.sync_copy(dst_ref.at[pl.ds(cbase, CH)],
                                dstv.at[pl.ds(0, CH)])
                for i in range(CH // 16):
                    sl = pl.ds(i * 16, 16)
                    gidx[sl] = sidx[sl] * NPL + p
                    eidx[sl] = pidx[sl] + p * EP
                cp1 = pltpu.async_copy(hm_ref.at[gidx], gbuf, sem)
                cp2 = pltpu.async_copy(emsg_ref.at[eidx], ebuf, sem2)
                cp1.wait()
                cp2.wait()

                r0 = jnp.maximum(lo - cbase, 0)
                r1 = jnp.minimum(hi - cbase, CH)

                def edge(rr, _):
                    dl = dstv[pl.ds(rr, 16)][0] - base_node
                    for j in range(W // 16):
                        sl = pl.ds(j * 16, 16)
                        g = gbuf[rr, sl]
                        e = ebuf[rr, sl]
                        if mode == 'relu_add':
                            m = jnp.maximum(g + e, 0.0)
                        else:
                            m = g * e
                        acc[dl, sl] = acc[dl, sl] + m
                    return 0
                lax.fori_loop(r0, r1, edge, 0)
                return 0
            lax.fori_loop(0, nch, chunk, 0)

            pltpu.sync_copy(acc, out_ref.at[p, pl.ds(base_node, slab)])

    return pl.kernel(
        body,
        out_type=jax.ShapeDtypeStruct((NPL, NP, W), jnp.float32),
        mesh=mesh,
        scratch_types=[
            pltpu.VMEM((slab, W), jnp.float32),
            pltpu.VMEM((CH, W), jnp.float32),
            pltpu.VMEM((CH, W), jnp.float32),
            pltpu.VMEM((CH,), jnp.int32),
            pltpu.VMEM((CH,), jnp.int32),
            pltpu.VMEM((CH,), jnp.int32),
            pltpu.VMEM((CH,), jnp.int32),
            pltpu.VMEM((CH + 16,), jnp.int32),
            pltpu.VMEM((48,), jnp.int32),
            pltpu.SemaphoreType.DMA,
            pltpu.SemaphoreType.DMA,
        ],
    )(hm2, emsg2, srcs, eids, dsts, bounds)


def kernel(x, edge_index, edge_attr, graph_ids, W_in, b_in, W_msg, b_msg,
           W_node, b_node, W_pn, W_pe, W_ps, W_r1, b_r1, W_r2, b_r2):
    N, DA = x.shape
    E, DB = edge_attr.shape
    D = W_in.shape[1]
    G = _N_GRAPHS
    n_layers = 4
    NPL = D // _W

    NP = _ceil_to(N, 256)
    EP = _ceil_to(E, 256)
    KA = _ceil_to(DA, 128)
    KB = _ceil_to(DB, 128)
    CH = min(128, EP)

    bf = jnp.bfloat16
    x_p = jnp.pad(x, ((0, NP - N), (0, KA - DA))).astype(bf)
    W_in_p = jnp.pad(W_in, ((0, KA - DA), (0, 0))).astype(bf)
    ea_p = jnp.pad(edge_attr, ((0, EP - E), (0, KB - DB))).astype(bf)
    Wm1 = W_msg[:D].astype(bf)
    Wm2_p = jnp.pad(W_msg[D:], ((0, KB - DB), (0, 0))).astype(bf)
    W_pe_p = jnp.pad(W_pe, ((0, KB - DB), (0, 0))).astype(bf)
    Wn1 = W_node[:D].astype(bf)
    Wn2 = W_node[D:].astype(bf)

    src = jnp.pad(edge_index[0], (0, EP - E), constant_values=0)
    dst = jnp.pad(edge_index[1], (0, EP - E), constant_values=NP - 1)
    gid_b = jnp.pad(graph_ids, (0, NP - N),
                    constant_values=G).reshape(NP // 256, 1, 256)

    n_tiles = _NC * _NS
    slab_n = NP // n_tiles
    perm0 = jnp.argsort(dst).astype(jnp.int32)
    dst_s = dst[perm0]
    src_s = src[perm0]
    bounds = jnp.searchsorted(
        dst_s, jnp.arange(n_tiles + 1, dtype=jnp.int32) * slab_n
    ).astype(jnp.int32)
    bounds = jnp.pad(bounds, (0, 48 - n_tiles - 1), constant_values=EP)
    srcs = jnp.pad(src_s, (0, CH))
    dsts = jnp.pad(dst_s, (0, CH), constant_values=NP - 1)
    eids = jnp.arange(EP + CH, dtype=jnp.int32)
    ea_s = ea_p[perm0]

    h = _dense_mm([(x_p, W_in_p, False, None)], bias=b_in, act=_relu, bk=KA,
                  out_dtype=bf)
    emsg_b = _dense_mm([(ea_s, Wm2_p, False, None)], bias=b_msg, bk=KB,
                       out_blocked=True)
    he_b = _dense_mm([(ea_s, W_pe_p, False, None)], bk=KB, out_blocked=True)

    emsg2 = emsg_b.reshape(-1, _W)
    he2 = he_b.reshape(-1, _W)
    for _ in range(n_layers):
        hm = _dense_mm([(h, Wm1, False, None)])
        n1 = _dense_mm([(h, Wn1, False, None)], out_dtype=bf)
        agg_b = _sc_edge(hm.reshape(-1, _W), emsg2, srcs, eids, dsts,
                         bounds, NP, EP, 'relu_add', CH=CH)
        h = _dense_mm([(agg_b, Wn2, True, None)], add=n1,
                      bias=b_node, act=_relu, out_dtype=bf)

    hv = _dense_mm([(h, W_pn.astype(bf), False, None)])
    hs = _dense_mm([(h, W_ps.astype(bf), False, None)])
    aggp_b = _sc_edge(hv.reshape(-1, _W), he2, srcs, eids, dsts,
                      bounds, NP, EP, 'mul', CH=CH)
    r = _dense_mm([(aggp_b, W_r1.astype(bf), True, hs)], bias=b_r1,
                  act=_leaky, out_dtype=bf)
    pooled = _pool_mean(r, gid_b, G)
    out = _dense_mm([(pooled, W_r2.astype(bf), False, None)], bias=b_r2,
                    bm=128)
    return out


# speedup vs baseline: 8.6834x; 1.0042x over previous
import jax
import jax.numpy as jnp
from jax import lax
from jax.experimental import pallas as pl
from jax.experimental.pallas import tpu as pltpu
from jax.experimental.pallas import tpu_sc as plsc

_N_GRAPHS = 128
_NC = 2
_NS = 16
_W = 256


def _ceil_to(a, b):
    return (a + b - 1) // b * b


def _relu(v):
    return jnp.maximum(v, 0.0)


def _leaky(v):
    return jnp.where(v > 0, v, 0.01 * v)


def _dense_mm(pairs, bias=None, add=None, mul=None, act=None, bm=1024,
              bn=1024, bk=512, out_blocked=False, out_dtype=jnp.float32):
    a0 = pairs[0][0]
    M = a0.shape[1] if pairs[0][2] else a0.shape[0]
    K = a0.shape[0] * 256 if pairs[0][2] else a0.shape[1]
    N = pairs[0][1].shape[1]
    bm = min(bm, M)
    bk = min(bk, K)
    if out_blocked:
        bn = 256
    bn = min(bn, N)
    nk = K // bk
    nsub = bk // 256
    has_bias = bias is not None
    has_add = add is not None
    has_mul = mul is not None

    def _bf(v):
        return v if v.dtype == jnp.bfloat16 else v.astype(jnp.bfloat16)

    def body(*refs):
        acc_ref = refs[-1]
        o_ref = refs[-2]
        k = pl.program_id(2)
        acc = jnp.zeros((bm, bn), jnp.float32)
        idx = 0
        for (_, _, blocked, amul) in pairs:
            a_ref = refs[idx]
            idx += 1
            b_ref = refs[idx]
            idx += 1
            if amul is not None:
                am_ref = refs[idx]
                idx += 1
            if blocked:
                b_all = b_ref[...]
                for q in range(nsub):
                    a = a_ref[q]
                    if amul is not None:
                        a = a * am_ref[..., q * 256:(q + 1) * 256]
                    acc = acc + jnp.dot(
                        _bf(a), _bf(b_all[q * 256:(q + 1) * 256, :]),
                        preferred_element_type=jnp.float32)
            else:
                a = a_ref[...]
                if amul is not None:
                    a = a * am_ref[...]
                acc = acc + jnp.dot(_bf(a), _bf(b_ref[...]),
                                    preferred_element_type=jnp.float32)
        n_in = idx

        @pl.when(k == 0)
        def _():
            acc_ref[...] = jnp.zeros((bm, bn), jnp.float32)

        acc_ref[...] += acc

        @pl.when(k == nk - 1)
        def _():
            v = acc_ref[...]
            i_extra = n_in
            if has_add:
                v = v + refs[i_extra][...]
                i_extra += 1
            if has_bias:
                v = v + refs[i_extra][0:1, :]
                i_extra += 1
            if act is not None:
                v = act(v)
            if has_mul:
                v = v * refs[i_extra][...]
            v = v.astype(out_dtype)
            if out_blocked:
                o_ref[0] = v
            else:
                o_ref[...] = v

    in_specs = []
    operands = []
    for (a, b, blocked, amul) in pairs:
        if blocked:
            in_specs.append(
                pl.BlockSpec((nsub, bm, 256), lambda i, j, k: (k, i, 0)))
        else:
            in_specs.append(pl.BlockSpec((bm, bk), lambda i, j, k: (i, k)))
        in_specs.append(pl.BlockSpec((bk, bn), lambda i, j, k: (k, j)))
        operands += [a, b]
        if amul is not None:
            in_specs.append(pl.BlockSpec((bm, bk), lambda i, j, k: (i, k)))
            operands.append(amul)
    if has_add:
        in_specs.append(pl.BlockSpec((bm, bn), lambda i, j, k: (i, j)))
        operands.append(add)
    if has_bias:
        in_specs.append(pl.BlockSpec((8, bn), lambda i, j, k: (0, j)))
        operands.append(jnp.broadcast_to(bias.reshape(1, -1), (8, N)))
    if has_mul:
        in_specs.append(pl.BlockSpec((bm, bn), lambda i, j, k: (i, j)))
        operands.append(mul)

    if out_blocked:
        out_spec = pl.BlockSpec((1, bm, 256), lambda i, j, k: (j, i, 0))
        out_shape = jax.ShapeDtypeStruct((N // 256, M, 256), out_dtype)
    else:
        out_spec = pl.BlockSpec((bm, bn), lambda i, j, k: (i, j))
        out_shape = jax.ShapeDtypeStruct((M, N), out_dtype)

    return pl.pallas_call(
        body,
        grid=(M // bm, N // bn, nk),
        in_specs=in_specs,
        out_specs=out_spec,
        out_shape=out_shape,
        scratch_shapes=[pltpu.VMEM((bm, bn), jnp.float32)],
    )(*operands)


def _pool_mean(r, gid_b, n_graphs, bn=256, bk=256):
    NP, N = r.shape
    bk = min(bk, NP)
    nk = NP // bk
    G = n_graphs

    def body(gid_ref, r_ref, o_ref, cnt_ref):
        k = pl.program_id(1)
        gid = gid_ref[0, 0, :]
        oh = (lax.broadcasted_iota(jnp.int32, (G, bk), 0) ==
              gid[None, :]).astype(jnp.float32)
        acc = jnp.dot(oh.astype(jnp.bfloat16), r_ref[...].astype(jnp.bfloat16),
                      preferred_element_type=jnp.float32)
        cnt = jnp.sum(oh, axis=1, keepdims=True)

        @pl.when(k == 0)
        def _():
            o_ref[...] = jnp.zeros_like(o_ref)
            cnt_ref[...] = jnp.zeros_like(cnt_ref)

        o_ref[...] += acc
        cnt_ref[...] += jnp.broadcast_to(cnt, cnt_ref.shape)

        @pl.when(k == nk - 1)
        def _():
            c = cnt_ref[...][:, 0:1]
            o_ref[...] = o_ref[...] / jnp.maximum(c, 1.0)

    return pl.pallas_call(
        body,
        grid=(N // bn, nk),
        in_specs=[pl.BlockSpec((1, 1, bk), lambda j, k: (k, 0, 0)),
                  pl.BlockSpec((bk, bn), lambda j, k: (k, j))],
        out_specs=pl.BlockSpec((G, bn), lambda j, k: (0, j)),
        out_shape=jax.ShapeDtypeStruct((G, N), jnp.float32),
        scratch_shapes=[pltpu.VMEM((G, 128), jnp.float32)],
    )(gid_b, r)


def _sc_edge(hm2, emsg2, srcs, eids, dsts, bounds, NP, EP, mode, CH=128,
             p0=0, npl=None):
    NPL = hm2.shape[0] // NP
    W = hm2.shape[1]
    if npl is None:
        npl = NPL
    n_tiles = _NC * _NS
    slab = NP // n_tiles

    mesh = plsc.VectorSubcoreMesh(core_axis_name="c", subcore_axis_name="s",
                                  num_cores=_NC, num_subcores=_NS)

    def body(hm_ref, emsg_ref, src_ref, eid_ref, dst_ref, bounds_ref,
             out_ref, acc, gbuf, ebuf, sidx, pidx, gidx, eidx, dstv,
             bnd, sem, sem2):
        c_id = lax.axis_index("c")
        s_id = lax.axis_index("s")
        wid = c_id * _NS + s_id
        base_node = wid * slab

        pltpu.sync_copy(bounds_ref, bnd)
        bv = bnd[pl.ds(wid, 16)]
        lo = bv[0]
        hi = bv[1]
        abase = (lo // 8) * 8
        nch = lax.div(hi - abase + CH - 1, CH)

        for p in range(npl):
            plane = p0 + p
            def zr(rr, _):
                for j in range(W // 16):
                    acc[rr, pl.ds(j * 16, 16)] = jnp.zeros((16,), jnp.float32)
                return 0
            lax.fori_loop(0, slab, zr, 0)

            def chunk(ch, _):
                cbase = abase + ch * CH
                pltpu.sync_copy(src_ref.at[pl.ds(cbase, CH)], sidx)
                pltpu.sync_copy(eid_ref.at[pl.ds(cbase, CH)], pidx)
                pltpu.sync_copy(dst_ref.at[pl.ds(cbase, CH)],
                                dstv.at[pl.ds(0, CH)])
                for i in range(CH // 16):
                    sl = pl.ds(i * 16, 16)
                    gidx[sl] = sidx[sl] * NPL + plane
                    eidx[sl] = pidx[sl] + plane * EP
                cp1 = pltpu.async_copy(hm_ref.at[gidx], gbuf, sem)
                cp2 = pltpu.async_copy(emsg_ref.at[eidx], ebuf, sem2)
                cp1.wait()
                cp2.wait()

                r0 = jnp.maximum(lo - cbase, 0)
                r1 = jnp.minimum(hi - cbase, CH)

                def edge(rr, _):
                    dl = dstv[pl.ds(rr, 16)][0] - base_node
                    for j in range(W // 16):
                        sl = pl.ds(j * 16, 16)
                        g = gbuf[rr, sl]
                        e = ebuf[rr, sl]
                        if mode == 'relu_add':
                            m = jnp.maximum(g + e, 0.0)
                        else:
                            m = g * e
                        acc[dl, sl] = acc[dl, sl] + m
                    return 0
                lax.fori_loop(r0, r1, edge, 0)
                return 0
            lax.fori_loop(0, nch, chunk, 0)

            pltpu.sync_copy(acc, out_ref.at[p, pl.ds(base_node, slab)])

    return pl.kernel(
        body,
        out_type=jax.ShapeDtypeStruct((npl, NP, W), jnp.float32),
        mesh=mesh,
        scratch_types=[
            pltpu.VMEM((slab, W), jnp.float32),
            pltpu.VMEM((CH, W), jnp.float32),
            pltpu.VMEM((CH, W), jnp.float32),
            pltpu.VMEM((CH,), jnp.int32),
            pltpu.VMEM((CH,), jnp.int32),
            pltpu.VMEM((CH,), jnp.int32),
            pltpu.VMEM((CH,), jnp.int32),
            pltpu.VMEM((CH + 16,), jnp.int32),
            pltpu.VMEM((48,), jnp.int32),
            pltpu.SemaphoreType.DMA,
            pltpu.SemaphoreType.DMA,
        ],
    )(hm2, emsg2, srcs, eids, dsts, bounds)


def kernel(x, edge_index, edge_attr, graph_ids, W_in, b_in, W_msg, b_msg,
           W_node, b_node, W_pn, W_pe, W_ps, W_r1, b_r1, W_r2, b_r2):
    N, DA = x.shape
    E, DB = edge_attr.shape
    D = W_in.shape[1]
    G = _N_GRAPHS
    n_layers = 4
    NPL = D // _W

    NP = _ceil_to(N, 256)
    EP = _ceil_to(E, 256)
    KA = _ceil_to(DA, 128)
    KB = _ceil_to(DB, 128)
    CH = min(128, EP)

    bf = jnp.bfloat16
    x_p = jnp.pad(x, ((0, NP - N), (0, KA - DA))).astype(bf)
    W_in_p = jnp.pad(W_in, ((0, KA - DA), (0, 0))).astype(bf)
    ea_p = jnp.pad(edge_attr, ((0, EP - E), (0, KB - DB))).astype(bf)
    Wm1 = W_msg[:D].astype(bf)
    Wm2_p = jnp.pad(W_msg[D:], ((0, KB - DB), (0, 0))).astype(bf)
    W_pe_p = jnp.pad(W_pe, ((0, KB - DB), (0, 0))).astype(bf)
    Wn1 = W_node[:D].astype(bf)
    Wn2 = W_node[D:].astype(bf)

    src = jnp.pad(edge_index[0], (0, EP - E), constant_values=0)
    dst = jnp.pad(edge_index[1], (0, EP - E), constant_values=NP - 1)
    gid_b = jnp.pad(graph_ids, (0, NP - N),
                    constant_values=G).reshape(NP // 256, 1, 256)

    n_tiles = _NC * _NS
    slab_n = NP // n_tiles
    perm0 = jnp.argsort(dst).astype(jnp.int32)
    dst_s = dst[perm0]
    src_s = src[perm0]
    bounds = jnp.searchsorted(
        dst_s, jnp.arange(n_tiles + 1, dtype=jnp.int32) * slab_n
    ).astype(jnp.int32)
    bounds = jnp.pad(bounds, (0, 48 - n_tiles - 1), constant_values=EP)
    srcs = jnp.pad(src_s, (0, CH))
    dsts = jnp.pad(dst_s, (0, CH), constant_values=NP - 1)
    eids = jnp.arange(EP + CH, dtype=jnp.int32)
    ea_s = ea_p[perm0]

    h = _dense_mm([(x_p, W_in_p, False, None)], bias=b_in, act=_relu, bk=KA,
                  out_dtype=bf)
    emsg_b = _dense_mm([(ea_s, Wm2_p, False, None)], bias=b_msg, bk=KB,
                       out_blocked=True)
    he_b = _dense_mm([(ea_s, W_pe_p, False, None)], bk=KB, out_blocked=True)

    emsg2 = emsg_b.reshape(-1, _W)
    he2 = he_b.reshape(-1, _W)
    half = NPL // 2
    hk = half * _W
    for _ in range(n_layers):
        hm = _dense_mm([(h, Wm1, False, None)])
        hm2 = hm.reshape(-1, _W)
        agg_a = _sc_edge(hm2, emsg2, srcs, eids, dsts,
                         bounds, NP, EP, 'relu_add', CH=CH, p0=0, npl=half)
        n1 = _dense_mm([(h, Wn1, False, None)], out_dtype=bf)
        agg_b = _sc_edge(hm2, emsg2, srcs, eids, dsts,
                         bounds, NP, EP, 'relu_add', CH=CH, p0=half,
                         npl=NPL - half)
        part = _dense_mm([(agg_a, Wn2[:hk], True, None)], add=n1)
        h = _dense_mm([(agg_b, Wn2[hk:], True, None)], add=part,
                      bias=b_node, act=_relu, out_dtype=bf)

    hv = _dense_mm([(h, W_pn.astype(bf), False, None)])
    hv2 = hv.reshape(-1, _W)
    aggp_a = _sc_edge(hv2, he2, srcs, eids, dsts,
                      bounds, NP, EP, 'mul', CH=CH, p0=0, npl=half)
    hs = _dense_mm([(h, W_ps.astype(bf), False, None)])
    aggp_b = _sc_edge(hv2, he2, srcs, eids, dsts,
                      bounds, NP, EP, 'mul', CH=CH, p0=half, npl=NPL - half)
    part_r = _dense_mm([(aggp_a, W_r1[:hk].astype(bf), True, hs[:, :hk])])
    r = _dense_mm([(aggp_b, W_r1[hk:].astype(bf), True, hs[:, hk:])],
                  add=part_r, bias=b_r1, act=_leaky, out_dtype=bf)
    pooled = _pool_mean(r, gid_b, G)
    out = _dense_mm([(pooled, W_r2.astype(bf), False, None)], bias=b_r2,
                    bm=128)
    return out
